# baseline (speedup 1.0000x reference)
import jax
import jax.numpy as jnp
from jax import lax
from jax.experimental import pallas as pl
from jax.experimental.pallas import tpu as pltpu
from jax.experimental.pallas import tpu_sc as plsc

_B, _C, _K, _L = 16, 144, 128, 256
_C_TIME = 128
_NC, _NS = 2, 16


def _tc_body(tab_ref, out_ref, pat_out, scr_time, scr_pat, sems, psem):
    ci = jax.lax.broadcasted_iota(jnp.int32, (_C_TIME, _L), 0)
    li = jax.lax.broadcasted_iota(jnp.int32, (_C_TIME, _L), 1)
    c_rem = ci - (ci // 2) * 2
    c_even = (ci - c_rem).astype(jnp.float32)
    ln10000 = 9.210340371976184
    div = jnp.exp(c_even * (-ln10000 / 128.0))
    angle = li.astype(jnp.float32) * div
    pe = jnp.where(c_rem == 0, jnp.sin(angle), jnp.cos(angle))
    tab_t = tab_ref[...].T
    scr_pat[...] = jnp.broadcast_to(tab_t[:, :, None], (_C - _C_TIME, _K, _L))
    pcopy = pltpu.make_async_copy(scr_pat, pat_out, psem)
    pcopy.start()
    scr_time[...] = jnp.broadcast_to(pe[:, None, :], (_C_TIME, _K, _L))
    copies = []
    for b in range(_B):
        copies.append(
            pltpu.make_async_copy(
                scr_time, out_ref.at[b, pl.ds(0, _C_TIME)], sems.at[b]
            )
        )
        copies[-1].start()
    pcopy.wait()
    for c in copies:
        c.wait()


def _tc_fill(table):
    return pl.pallas_call(
        _tc_body,
        in_specs=[pl.BlockSpec((_K, _C - _C_TIME), lambda: (0, 0))],
        out_specs=[
            pl.BlockSpec(memory_space=pl.ANY),
            pl.BlockSpec(memory_space=pl.ANY),
        ],
        out_shape=[
            jax.ShapeDtypeStruct((_B, _C, _K, _L), jnp.float32),
            jax.ShapeDtypeStruct((_C - _C_TIME, _K, _L), jnp.float32),
        ],
        scratch_shapes=[
            pltpu.VMEM((_C_TIME, _K, _L), jnp.float32),
            pltpu.VMEM((_C - _C_TIME, _K, _L), jnp.float32),
            pltpu.SemaphoreType.DMA((_B,)),
            pltpu.SemaphoreType.DMA,
        ],
    )(table)


def _sc_body(pat_hbm, out_hbm_ref, pat_v, sem):
    wid = lax.axis_index("s") * _NC + lax.axis_index("c")
    j = wid % 16
    bhalf = wid // 16
    pltpu.sync_copy(pat_hbm.at[j], pat_v)
    copies = []
    for i in range(8):
        b = bhalf * 8 + i
        copies.append(
            pltpu.async_copy(pat_v, out_hbm_ref.at[b, _C_TIME + j], sem)
        )
    for c in copies:
        c.wait()


def _sc_fill_inplace(patterns, big_ref):
    mesh = plsc.VectorSubcoreMesh(
        core_axis_name="c", subcore_axis_name="s",
        num_cores=_NC, num_subcores=_NS,
    )
    f = pl.kernel(
        _sc_body,
        out_type=(),
        mesh=mesh,
        scratch_types=[
            pltpu.VMEM((_K, _L), jnp.float32),
            pltpu.SemaphoreType.DMA,
        ],
    )
    f(patterns, big_ref)


def kernel(cond_mask, table):
    del cond_mask
    big, patterns = _tc_fill(table)
    ref = jax.new_ref(big)
    _sc_fill_inplace(patterns, ref)
    return ref[...]
